# baseline (speedup 1.0000x reference)
import functools

import jax
import jax.numpy as jnp
from jax import lax
from jax.experimental import pallas as pl
from jax.experimental.pallas import tpu as pltpu
from jax.experimental.pallas import tpu_sc as plsc

N = 100000
G = 256
D_IN = 128
D_H = 64
H = 4
B = 2048
NPAD = 100352
NB = NPAD // B
W = 72

NTILES = 32
C = NPAD // NTILES


def _window(seg_ref):
    g0 = jnp.minimum(jnp.minimum(seg_ref[0, 0, 0], G - 1) & ~7, G - W)
    span_ok = (seg_ref[0, 0, B - 1] - g0) < W
    return g0, span_ok


def _ohT(seg_row, g0, width):
    return (lax.broadcasted_iota(jnp.int32, (width, B), 0) + g0
            == seg_row).astype(jnp.float32)


def _k1_body(h_ref, w1t_ref, b1_ref, w2t_ref, gmb_ref, e_ref):
    i = pl.program_id(0)
    z = jnp.tanh(jnp.dot(h_ref[...], w1t_ref[...],
                         preferred_element_type=jnp.float32) + b1_ref[...])
    st = lax.dot_general(w2t_ref[...], z, (((0,), (1,)), ((), ())),
                         preferred_element_type=jnp.float32)
    col = lax.broadcasted_iota(jnp.int32, (H, B), 1) + i * B
    e_ref[...] = jnp.where(col < N, jnp.exp(st - gmb_ref[...]),
                           0.0).reshape(1, H, B)


def _sc_body(e0_hbm, e1_hbm, e2_hbm, e3_hbm, seg_hbm, w_hbm,
             est, seg_s, ec0, ec1, ec2, ec3, seg_c, dv0, dv1, dv2, dv3,
             wv, zbuf, acc0, acc1, acc2, acc3):
    ehs = [e0_hbm, e1_hbm, e2_hbm, e3_hbm]
    ec = [ec0, ec1, ec2, ec3]
    dv = [dv0, dv1, dv2, dv3]
    c = lax.axis_index("c")
    s = lax.axis_index("s")
    acc = [acc0, acc1, acc2, acc3]

    @pl.when(s == 0)
    def _():
        for j in range((G + 8) // 16):
            zbuf[pl.ds(j * 16, 16)] = jnp.zeros((16,), jnp.float32)
        for hh in range(H):
            pltpu.sync_copy(zbuf, acc[hh])

    plsc.subcore_barrier()

    for k in range(2):
        base = (s + 16 * k) * C
        pltpu.sync_copy(seg_hbm.at[pl.ds(base, C)], seg_s)
        for hh in range(H):
            pltpu.sync_copy(ehs[hh].at[pl.ds(base, C)], est)
            pltpu.sync_copy(est, acc[hh].at[seg_s], add=True)

    plsc.subcore_barrier()

    base2 = (s + 16 * c) * C
    pltpu.sync_copy(seg_hbm.at[pl.ds(base2, C)], seg_c)
    for hh in range(H):
        pltpu.sync_copy(ehs[hh].at[pl.ds(base2, C)], ec[hh])
        pltpu.sync_copy(acc[hh].at[seg_c], dv[hh])

    def body(j, _):
        sl = pl.ds(j * 16, 16)
        w16 = jnp.zeros((16,), jnp.float32)
        for hh in range(H):
            w16 = w16 + ec[hh][sl] / (dv[hh][sl] + 1e-12)
        wv[sl] = w16 * (1.0 / H)
        return 0

    lax.fori_loop(0, C // 16, body, 0)
    pltpu.sync_copy(wv, w_hbm.at[pl.ds(base2, C)])


def _k2_body(h_ref, w_ref, seg_ref, out_ref, pool_s):
    i = pl.program_id(0)
    seg_row = seg_ref[0, :, :]
    w_row = w_ref[0, :, :]
    g0, span_ok = _window(seg_ref)
    row = lax.broadcasted_iota(jnp.int32, (B, 1), 0) + i * B
    hb = jnp.where(row < N, h_ref[...], 0.0).astype(jnp.bfloat16)

    @pl.when(i == 0)
    def _():
        pool_s[...] = jnp.zeros((G, D_IN), jnp.float32)

    @pl.when(span_ok)
    def _():
        ohTw = _ohT(seg_row, g0, W).astype(jnp.bfloat16) \
            * w_row.astype(jnp.bfloat16)
        pool_s[pl.ds(g0, W), :] += lax.dot_general(
            ohTw, hb, (((1,), (0,)), ((), ())),
            preferred_element_type=jnp.float32)

    @pl.when(jnp.logical_not(span_ok))
    def _():
        ohTw = _ohT(seg_row, 0, G).astype(jnp.bfloat16) \
            * w_row.astype(jnp.bfloat16)
        pool_s[...] += lax.dot_general(
            ohTw, hb, (((1,), (0,)), ((), ())),
            preferred_element_type=jnp.float32)

    @pl.when(i == NB - 1)
    def _():
        out_ref[...] = pool_s[...]


def kernel(h, segment_ids, fc1_w, fc1_b, fc2_w, fc2_b):
    seg = segment_ids.astype(jnp.int32)
    segp = jnp.pad(seg, (0, NPAD - N), constant_values=G)
    seg3d = segp.reshape(NB, 1, B)
    w1t = fc1_w.T
    w2t = fc2_w.T
    b1 = fc1_b.reshape(1, D_H)
    gmb = jnp.sum(jnp.abs(fc2_w), axis=1).reshape(H, 1)

    e = pl.pallas_call(
        _k1_body,
        grid=(NB,),
        in_specs=[
            pl.BlockSpec((B, D_IN), lambda i: (i, 0)),
            pl.BlockSpec((D_IN, D_H), lambda i: (0, 0)),
            pl.BlockSpec((1, D_H), lambda i: (0, 0)),
            pl.BlockSpec((D_H, H), lambda i: (0, 0)),
            pl.BlockSpec((H, 1), lambda i: (0, 0)),
        ],
        out_specs=pl.BlockSpec((1, H, B), lambda i: (i, 0, 0)),
        out_shape=jax.ShapeDtypeStruct((NB, H, B), jnp.float32),
    )(h, w1t, b1, w2t, gmb)

    ehead = [e[:, hh, :].reshape(NPAD) for hh in range(H)]

    mesh = plsc.VectorSubcoreMesh(core_axis_name="c", subcore_axis_name="s")
    sc = functools.partial(
        pl.kernel,
        mesh=mesh,
        out_type=jax.ShapeDtypeStruct((NPAD,), jnp.float32),
        scratch_types=[
            pltpu.VMEM((C,), jnp.float32),
            pltpu.VMEM((C,), jnp.int32),
            pltpu.VMEM((C,), jnp.float32),
            pltpu.VMEM((C,), jnp.float32),
            pltpu.VMEM((C,), jnp.float32),
            pltpu.VMEM((C,), jnp.float32),
            pltpu.VMEM((C,), jnp.int32),
            pltpu.VMEM((C,), jnp.float32),
            pltpu.VMEM((C,), jnp.float32),
            pltpu.VMEM((C,), jnp.float32),
            pltpu.VMEM((C,), jnp.float32),
            pltpu.VMEM((C,), jnp.float32),
            pltpu.VMEM((G + 8,), jnp.float32),
            pltpu.VMEM_SHARED((G + 8,), jnp.float32),
            pltpu.VMEM_SHARED((G + 8,), jnp.float32),
            pltpu.VMEM_SHARED((G + 8,), jnp.float32),
            pltpu.VMEM_SHARED((G + 8,), jnp.float32),
        ],
    )(_sc_body)
    w = sc(*ehead, segp)

    w3d = w.reshape(NB, 1, B)
    out = pl.pallas_call(
        _k2_body,
        grid=(NB,),
        in_specs=[
            pl.BlockSpec((B, D_IN), lambda i: (i, 0)),
            pl.BlockSpec((1, 1, B), lambda i: (i, 0, 0)),
            pl.BlockSpec((1, 1, B), lambda i: (i, 0, 0)),
        ],
        out_specs=pl.BlockSpec((G, D_IN), lambda i: (0, 0)),
        out_shape=jax.ShapeDtypeStruct((G, D_IN), jnp.float32),
        scratch_shapes=[pltpu.VMEM((G, D_IN), jnp.float32)],
    )(h, w3d, seg3d)

    return out


# speedup vs baseline: 2.3413x; 2.3413x over previous
import jax
import jax.numpy as jnp
from jax import lax
from jax.experimental import pallas as pl
from jax.experimental.pallas import tpu as pltpu

N = 100000
G = 256
D_IN = 128
D_H = 64
H = 4
B = 5000
NB = N // B
W = 72

_NEG = -1e30


def _window(seg_ref):
    g0 = jnp.minimum(seg_ref[0, 0, 0] & ~7, G - W)
    span_ok = (seg_ref[0, 0, B - 1] - g0) < W
    return g0, span_ok


def _ohT(seg_row, g0, width):
    return (lax.broadcasted_iota(jnp.int32, (width, B), 0) + g0
            == seg_row).astype(jnp.float32)


def _body(h_ref, w1t_ref, b1_ref, w2t_ref, gmb_ref, seg_ref, out_ref,
          hbf_s, st_s, acc_s, rssum_s, pool_s):
    p = pl.program_id(0)
    i = pl.program_id(1)
    seg_row = seg_ref[0, :, :]
    g0, span_ok = _window(seg_ref)

    @pl.when(p == 0)
    def _phase_a():
        z = jnp.tanh(jnp.dot(h_ref[...], w1t_ref[...],
                             preferred_element_type=jnp.float32) + b1_ref[...])
        st = lax.dot_general(w2t_ref[...], z, (((0,), (1,)), ((), ())),
                             preferred_element_type=jnp.float32)
        st_s[i] = st
        hbf_s[i] = h_ref[...].astype(jnp.bfloat16)

        @pl.when(i == 0)
        def _():
            acc_s[...] = jnp.zeros((G, H), jnp.float32)

        e_t = jnp.exp(st - gmb_ref[...])

        @pl.when(span_ok)
        def _():
            ohT = _ohT(seg_row, g0, W)
            ps = lax.dot_general(e_t, ohT, (((1,), (1,)), ((), ())),
                                 preferred_element_type=jnp.float32)
            acc_s[pl.ds(g0, W), :] += jnp.transpose(ps)

        @pl.when(jnp.logical_not(span_ok))
        def _():
            ohT = _ohT(seg_row, 0, G)
            ps = lax.dot_general(e_t, ohT, (((1,), (1,)), ((), ())),
                                 preferred_element_type=jnp.float32)
            acc_s[...] += jnp.transpose(ps)

        @pl.when(i == NB - 1)
        def _():
            rssum_s[...] = 1.0 / (acc_s[...] + 1e-12)

    @pl.when(p == 1)
    def _phase_b():
        e_t = jnp.exp(st_s[i] - gmb_ref[...])
        hb = hbf_s[i]

        @pl.when(i == 0)
        def _():
            pool_s[...] = jnp.zeros((G, D_IN), jnp.float32)

        @pl.when(span_ok)
        def _():
            ohT = _ohT(seg_row, g0, W)
            rd_t = lax.dot_general(rssum_s[pl.ds(g0, W), :], ohT,
                                   (((0,), (0,)), ((), ())),
                                   preferred_element_type=jnp.float32)
            w_t = jnp.sum(e_t * rd_t, axis=0, keepdims=True) * (1.0 / H)
            ohTw = ohT.astype(jnp.bfloat16) * w_t.astype(jnp.bfloat16)
            pool_s[pl.ds(g0, W), :] += lax.dot_general(
                ohTw, hb, (((1,), (0,)), ((), ())),
                preferred_element_type=jnp.float32)

        @pl.when(jnp.logical_not(span_ok))
        def _():
            ohT = _ohT(seg_row, 0, G)
            rd_t = lax.dot_general(rssum_s[...], ohT,
                                   (((0,), (0,)), ((), ())),
                                   preferred_element_type=jnp.float32)
            w_t = jnp.sum(e_t * rd_t, axis=0, keepdims=True) * (1.0 / H)
            ohTw = ohT.astype(jnp.bfloat16) * w_t.astype(jnp.bfloat16)
            pool_s[...] += lax.dot_general(
                ohTw, hb, (((1,), (0,)), ((), ())),
                preferred_element_type=jnp.float32)

        @pl.when(i == NB - 1)
        def _():
            out_ref[...] = pool_s[...]


def kernel(h, segment_ids, fc1_w, fc1_b, fc2_w, fc2_b):
    seg3d = segment_ids.astype(jnp.int32).reshape(NB, 1, B)
    w1t = fc1_w.T
    w2t = fc2_w.T
    b1 = fc1_b.reshape(1, D_H)
    gmb = jnp.sum(jnp.abs(fc2_w), axis=1).reshape(H, 1)

    out = pl.pallas_call(
        _body,
        grid=(2, NB),
        in_specs=[
            pl.BlockSpec((B, D_IN), lambda p, i: (i * (1 - p), 0)),
            pl.BlockSpec((D_IN, D_H), lambda p, i: (0, 0)),
            pl.BlockSpec((1, D_H), lambda p, i: (0, 0)),
            pl.BlockSpec((D_H, H), lambda p, i: (0, 0)),
            pl.BlockSpec((H, 1), lambda p, i: (0, 0)),
            pl.BlockSpec((1, 1, B), lambda p, i: (i, 0, 0)),
        ],
        out_specs=pl.BlockSpec((G, D_IN), lambda p, i: (0, 0)),
        out_shape=jax.ShapeDtypeStruct((G, D_IN), jnp.float32),
        scratch_shapes=[
            pltpu.VMEM((NB, B, D_IN), jnp.bfloat16),
            pltpu.VMEM((NB, H, B), jnp.float32),
            pltpu.VMEM((G, H), jnp.float32),
            pltpu.VMEM((G, H), jnp.float32),
            pltpu.VMEM((G, D_IN), jnp.float32),
        ],
    )(h, w1t, b1, w2t, gmb, seg3d)

    return out
